# baseline (speedup 1.0000x reference)
import jax
import jax.numpy as jnp
from jax import lax
from jax.experimental import pallas as pl
from jax.experimental.pallas import tpu as pltpu
from jax.experimental.pallas import tpu_sc as plsc

N = 10000
NNZ = 320000
E = 5000
D = 128

NC = 2
NS = 16
NW = NC * NS
PER_W = NNZ // NW
CHUNK = 40
STEPS = PER_W // CHUNK
NBUF = 5
PSTEPS = 256
CCHUNK = 40
CSTEPS = PER_W // CCHUNK
CNB = 5

_MESH = plsc.VectorSubcoreMesh(core_axis_name="c", subcore_axis_name="s")


def _fill_ones(ref):
    one = jnp.full((16,), 1.0, jnp.float32)
    nrows = ref.shape[0]

    def body(i, _):
        r = i // (D // 16)
        q = (i % (D // 16)) * 16
        ref[r, pl.ds(q, 16)] = one
        return 0

    lax.fori_loop(0, nrows * (D // 16), body, 0)


def _feature_body(src_hbm, gidx_hbm, sidx_hbm, z_hbm, out_hbm,
                  rows, sems, gidx, sidx, acc):
    c = lax.axis_index("c")
    s = lax.axis_index("s")
    wid = c * NS + s
    base0 = wid * PER_W

    pltpu.sync_copy(gidx_hbm.at[pl.ds(base0, PER_W)], gidx)

    for b in range(NBUF):
        off = base0 + b * CHUNK
        pltpu.async_copy(sidx_hbm.at[pl.ds(off, CHUNK)], sidx[b], sems[b])
        pltpu.async_copy(src_hbm.at[gidx.at[pl.ds(b * CHUNK, CHUNK)]],
                         rows[b], sems[b])

    @pl.when(s == 0)
    def _():
        pltpu.sync_copy(z_hbm, acc)

    plsc.subcore_barrier()

    def outer(it, _):
        jj = it * NBUF
        for b in range(NBUF):
            j = jj + b
            pltpu.make_async_copy(
                sidx_hbm.at[pl.ds(base0, CHUNK)], sidx[b], sems[b]).wait()
            pltpu.make_async_copy(
                src_hbm.at[gidx.at[pl.ds(0, CHUNK)]], rows[b], sems[b]).wait()
            pltpu.sync_copy(rows[b], acc.at[sidx[b]], add=True)

            @pl.when(j + NBUF < STEPS)
            def _():
                off = base0 + (j + NBUF) * CHUNK
                pltpu.async_copy(sidx_hbm.at[pl.ds(off, CHUNK)], sidx[b], sems[b])
                pltpu.async_copy(
                    src_hbm.at[gidx.at[pl.ds((j + NBUF) * CHUNK, CHUNK)]],
                    rows[b], sems[b])
        return 0

    lax.fori_loop(0, STEPS // NBUF, outer, 0)

    plsc.subcore_barrier()

    @pl.when(s == 0)
    def _():
        pltpu.sync_copy(acc, out_hbm.at[c])


def _phase1_body(x_hbm, vtx_hbm, edg_hbm, z_ed, xe_out, *scr):
    rows = list(scr[0:NBUF])
    sems = list(scr[NBUF:2 * NBUF])
    gidx = scr[2 * NBUF]
    sidx = list(scr[2 * NBUF + 1:2 * NBUF + 1 + NBUF])
    acc = scr[2 * NBUF + 1 + NBUF]
    _feature_body(x_hbm, vtx_hbm, edg_hbm, z_ed, xe_out,
                  rows, sems, gidx, sidx, acc)


def _phase2_body(xe_hbm, vtx_hbm, edg_hbm, z_nd, xv_out, *scr):
    rows = list(scr[0:NBUF])
    sems = list(scr[NBUF:2 * NBUF])
    gidx = scr[2 * NBUF]
    sidx = list(scr[2 * NBUF + 1:2 * NBUF + 1 + NBUF])
    acc = scr[2 * NBUF + 1 + NBUF]
    _feature_body(xe_hbm, edg_hbm, vtx_hbm, z_nd, xv_out,
                  rows, sems, gidx, sidx, acc)


def _feat_scratch(nseg):
    return (
        [pltpu.VMEM((CHUNK, D), jnp.float32) for _ in range(NBUF)]
        + [pltpu.SemaphoreType.DMA for _ in range(NBUF)]
        + [pltpu.VMEM((PER_W,), jnp.int32)]
        + [pltpu.VMEM((CHUNK,), jnp.int32) for _ in range(NBUF)]
        + [pltpu.VMEM_SHARED((nseg, D), jnp.float32)]
    )


_phase1 = pl.kernel(
    _phase1_body,
    out_type=jax.ShapeDtypeStruct((NC, E, D), jnp.float32),
    mesh=_MESH,
    scratch_types=_feat_scratch(E),
)

_phase2 = pl.kernel(
    _phase2_body,
    out_type=jax.ShapeDtypeStruct((NC, N, D), jnp.float32),
    mesh=_MESH,
    scratch_types=_feat_scratch(N),
)


def _counts_body(vtx_hbm, edg_hbm, z_ed, z_nd, ce_out, cv_out, *scr):
    vbufs = list(scr[0:CNB])
    ebufs = list(scr[CNB:2 * CNB])
    vsems = list(scr[2 * CNB:3 * CNB])
    esems = list(scr[3 * CNB:4 * CNB])
    ones, acc_ce, acc_cv = scr[4 * CNB:]

    c = lax.axis_index("c")
    s = lax.axis_index("s")
    wid = c * NS + s
    base0 = wid * PER_W

    _fill_ones(ones)

    for b in range(CNB):
        off = base0 + b * CCHUNK
        pltpu.async_copy(vtx_hbm.at[pl.ds(off, CCHUNK)], vbufs[b], vsems[b])
        pltpu.async_copy(edg_hbm.at[pl.ds(off, CCHUNK)], ebufs[b], esems[b])

    @pl.when(s == 0)
    def _():
        pltpu.sync_copy(z_ed, acc_ce)
        pltpu.sync_copy(z_nd, acc_cv)

    plsc.subcore_barrier()

    def outer(it, _):
        jj = it * CNB
        for b in range(CNB):
            j = jj + b
            pltpu.make_async_copy(
                vtx_hbm.at[pl.ds(base0, CCHUNK)], vbufs[b], vsems[b]).wait()
            pltpu.make_async_copy(
                edg_hbm.at[pl.ds(base0, CCHUNK)], ebufs[b], esems[b]).wait()
            pltpu.sync_copy(ones, acc_cv.at[vbufs[b]], add=True)
            pltpu.sync_copy(ones, acc_ce.at[ebufs[b]], add=True)

            @pl.when(j + CNB < CSTEPS)
            def _():
                off = base0 + (j + CNB) * CCHUNK
                pltpu.async_copy(vtx_hbm.at[pl.ds(off, CCHUNK)], vbufs[b], vsems[b])
                pltpu.async_copy(edg_hbm.at[pl.ds(off, CCHUNK)], ebufs[b], esems[b])
        return 0

    lax.fori_loop(0, CSTEPS // CNB, outer, 0)

    plsc.subcore_barrier()

    @pl.when(s == 0)
    def _():
        pltpu.sync_copy(acc_ce, ce_out.at[c])
        pltpu.sync_copy(acc_cv, cv_out.at[c])


_counts = pl.kernel(
    _counts_body,
    out_type=(
        jax.ShapeDtypeStruct((NC, E, D), jnp.float32),
        jax.ShapeDtypeStruct((NC, N, D), jnp.float32),
    ),
    mesh=_MESH,
    scratch_types=(
        [pltpu.VMEM((CCHUNK,), jnp.int32) for _ in range(2 * CNB)]
        + [pltpu.SemaphoreType.DMA for _ in range(2 * CNB)]
        + [
            pltpu.VMEM((CCHUNK, D), jnp.float32),
            pltpu.VMEM_SHARED((E, D), jnp.float32),
            pltpu.VMEM_SHARED((N, D), jnp.float32),
        ]
    ),
)


def _norm_body(xe_ref, ce_ref, out_ref):
    cnt = ce_ref[0, :, 0] + ce_ref[1, :, 0]
    s = xe_ref[0] + xe_ref[1]
    out_ref[...] = s / jnp.clip(cnt, 1.0)[:, None]


_BE = 1000


def _normalize(xe_parts, ce_parts):
    return pl.pallas_call(
        _norm_body,
        out_shape=jax.ShapeDtypeStruct((E, D), jnp.float32),
        grid=(E // _BE,),
        in_specs=[
            pl.BlockSpec((NC, _BE, D), lambda i: (0, i, 0)),
            pl.BlockSpec((NC, _BE, D), lambda i: (0, i, 0)),
        ],
        out_specs=pl.BlockSpec((_BE, D), lambda i: (i, 0)),
    )(xe_parts, ce_parts)


def _tail_body(xv_ref, cv_ref, x0_ref, w_ref, ab_ref, out_ref):
    a = ab_ref[0, 0]
    b = ab_ref[0, 1]
    cnt = cv_ref[0, :, 0] + cv_ref[1, :, 0]
    xv = (xv_ref[0] + xv_ref[1]) / jnp.clip(cnt, 1.0)[:, None]
    xi = (1.0 - a) * xv + a * x0_ref[...]
    out_ref[...] = (1.0 - b) * xi + b * jnp.dot(
        xi, w_ref[...].T, preferred_element_type=jnp.float32)


_BN = 1000


def _tail(xv_parts, cv_parts, x0, w, ab):
    return pl.pallas_call(
        _tail_body,
        out_shape=jax.ShapeDtypeStruct((N, D), jnp.float32),
        grid=(N // _BN,),
        in_specs=[
            pl.BlockSpec((NC, _BN, D), lambda i: (0, i, 0)),
            pl.BlockSpec((NC, _BN, D), lambda i: (0, i, 0)),
            pl.BlockSpec((_BN, D), lambda i: (i, 0)),
            pl.BlockSpec((D, D), lambda i: (0, 0)),
            pl.BlockSpec(memory_space=pltpu.SMEM),
        ],
        out_specs=pl.BlockSpec((_BN, D), lambda i: (i, 0)),
    )(xv_parts, cv_parts, x0, w, ab)


def kernel(X, vertex, edges, alpha, beta, X0, W):
    vertex = vertex.astype(jnp.int32)
    edges = edges.astype(jnp.int32)
    z_ed = jnp.zeros((E, D), jnp.float32)
    z_nd = jnp.zeros((N, D), jnp.float32)

    xe_p = _phase1(X, vertex, edges, z_ed)
    ce_p, cv_p = _counts(vertex, edges, z_ed, z_nd)
    xe = _normalize(xe_p, ce_p)
    xv_p = _phase2(xe, vertex, edges, z_nd)
    ab = jnp.stack([alpha, beta]).astype(jnp.float32).reshape(1, 2)
    return _tail(xv_p, cv_p, X0, W, ab)


# speedup vs baseline: 1.0364x; 1.0364x over previous
import jax
import jax.numpy as jnp
from jax import lax
from jax.experimental import pallas as pl
from jax.experimental.pallas import tpu as pltpu
from jax.experimental.pallas import tpu_sc as plsc

N = 10000
NNZ = 320000
E = 5000
D = 128

NC = 2
NS = 16
NW = NC * NS
PER_W = NNZ // NW
CHUNK = 40
STEPS = PER_W // CHUNK
NBUF = 5
PSTEPS = 256
CCHUNK = 80
CSTEPS = PER_W // CCHUNK
CNB = 5

_MESH = plsc.VectorSubcoreMesh(core_axis_name="c", subcore_axis_name="s")


def _fill_ones(ref):
    one = jnp.full((16,), 1.0, jnp.float32)
    nrows = ref.shape[0]

    def body(i, _):
        r = i // (D // 16)
        q = (i % (D // 16)) * 16
        ref[r, pl.ds(q, 16)] = one
        return 0

    lax.fori_loop(0, nrows * (D // 16), body, 0)


def _feature_body(src_hbm, gidx_hbm, sidx_hbm, z_hbm, out_hbm,
                  rows, sems, gidx, sidx, acc):
    c = lax.axis_index("c")
    s = lax.axis_index("s")
    wid = c * NS + s
    base0 = wid * PER_W

    pltpu.sync_copy(gidx_hbm.at[pl.ds(base0, PER_W)], gidx)

    for b in range(NBUF):
        off = base0 + b * CHUNK
        pltpu.async_copy(sidx_hbm.at[pl.ds(off, CHUNK)], sidx[b], sems[b])
        pltpu.async_copy(src_hbm.at[gidx.at[pl.ds(b * CHUNK, CHUNK)]],
                         rows[b], sems[b])

    @pl.when(s == 0)
    def _():
        pltpu.sync_copy(z_hbm, acc)

    plsc.subcore_barrier()

    def outer(it, _):
        jj = it * NBUF
        for b in range(NBUF):
            j = jj + b
            pltpu.make_async_copy(
                sidx_hbm.at[pl.ds(base0, CHUNK)], sidx[b], sems[b]).wait()
            pltpu.make_async_copy(
                src_hbm.at[gidx.at[pl.ds(0, CHUNK)]], rows[b], sems[b]).wait()
            pltpu.sync_copy(rows[b], acc.at[sidx[b]], add=True)

            @pl.when(j + NBUF < STEPS)
            def _():
                off = base0 + (j + NBUF) * CHUNK
                pltpu.async_copy(sidx_hbm.at[pl.ds(off, CHUNK)], sidx[b], sems[b])
                pltpu.async_copy(
                    src_hbm.at[gidx.at[pl.ds((j + NBUF) * CHUNK, CHUNK)]],
                    rows[b], sems[b])
        return 0

    lax.fori_loop(0, STEPS // NBUF, outer, 0)

    plsc.subcore_barrier()

    @pl.when(s == 0)
    def _():
        pltpu.sync_copy(acc, out_hbm.at[c])


def _phase1_body(x_hbm, vtx_hbm, edg_hbm, z_ed, xe_out, *scr):
    rows = list(scr[0:NBUF])
    sems = list(scr[NBUF:2 * NBUF])
    gidx = scr[2 * NBUF]
    sidx = list(scr[2 * NBUF + 1:2 * NBUF + 1 + NBUF])
    acc = scr[2 * NBUF + 1 + NBUF]
    _feature_body(x_hbm, vtx_hbm, edg_hbm, z_ed, xe_out,
                  rows, sems, gidx, sidx, acc)


def _phase2_body(xe_hbm, vtx_hbm, edg_hbm, z_nd, xv_out, *scr):
    rows = list(scr[0:NBUF])
    sems = list(scr[NBUF:2 * NBUF])
    gidx = scr[2 * NBUF]
    sidx = list(scr[2 * NBUF + 1:2 * NBUF + 1 + NBUF])
    acc = scr[2 * NBUF + 1 + NBUF]
    _feature_body(xe_hbm, edg_hbm, vtx_hbm, z_nd, xv_out,
                  rows, sems, gidx, sidx, acc)


def _feat_scratch(nseg):
    return (
        [pltpu.VMEM((CHUNK, D), jnp.float32) for _ in range(NBUF)]
        + [pltpu.SemaphoreType.DMA for _ in range(NBUF)]
        + [pltpu.VMEM((PER_W,), jnp.int32)]
        + [pltpu.VMEM((CHUNK,), jnp.int32) for _ in range(NBUF)]
        + [pltpu.VMEM_SHARED((nseg, D), jnp.float32)]
    )


_phase1 = pl.kernel(
    _phase1_body,
    out_type=jax.ShapeDtypeStruct((NC, E, D), jnp.float32),
    mesh=_MESH,
    scratch_types=_feat_scratch(E),
)

_phase2 = pl.kernel(
    _phase2_body,
    out_type=jax.ShapeDtypeStruct((NC, N, D), jnp.float32),
    mesh=_MESH,
    scratch_types=_feat_scratch(N),
)


def _count_body(idx_hbm, z_hbm, cnt_out, *scr):
    bufs = list(scr[0:CNB])
    sems = list(scr[CNB:2 * CNB])
    ones, acc = scr[2 * CNB:]

    c = lax.axis_index("c")
    s = lax.axis_index("s")
    wid = c * NS + s
    base0 = wid * PER_W

    _fill_ones(ones)

    for b in range(CNB):
        off = base0 + b * CCHUNK
        pltpu.async_copy(idx_hbm.at[pl.ds(off, CCHUNK)], bufs[b], sems[b])

    @pl.when(s == 0)
    def _():
        pltpu.sync_copy(z_hbm, acc)

    plsc.subcore_barrier()

    def outer(it, _):
        jj = it * CNB
        for b in range(CNB):
            j = jj + b
            pltpu.make_async_copy(
                idx_hbm.at[pl.ds(base0, CCHUNK)], bufs[b], sems[b]).wait()
            pltpu.sync_copy(ones, acc.at[bufs[b]], add=True)

            @pl.when(j + CNB < CSTEPS)
            def _():
                off = base0 + (j + CNB) * CCHUNK
                pltpu.async_copy(idx_hbm.at[pl.ds(off, CCHUNK)], bufs[b], sems[b])
        return 0

    lax.fori_loop(0, CSTEPS // CNB, outer, 0)

    plsc.subcore_barrier()

    @pl.when(s == 0)
    def _():
        pltpu.sync_copy(acc, cnt_out.at[c])


def _make_count(nseg):
    return pl.kernel(
        _count_body,
        out_type=jax.ShapeDtypeStruct((NC, nseg, D), jnp.float32),
        mesh=_MESH,
        scratch_types=(
            [pltpu.VMEM((CCHUNK,), jnp.int32) for _ in range(CNB)]
            + [pltpu.SemaphoreType.DMA for _ in range(CNB)]
            + [
                pltpu.VMEM((CCHUNK, D), jnp.float32),
                pltpu.VMEM_SHARED((nseg, D), jnp.float32),
            ]
        ),
    )


_ecount = _make_count(E)
_vcount = _make_count(N)


def _norm_body(xe_ref, ce_ref, out_ref):
    cnt = ce_ref[0, :, 0] + ce_ref[1, :, 0]
    s = xe_ref[0] + xe_ref[1]
    out_ref[...] = s / jnp.clip(cnt, 1.0)[:, None]


_BE = 1000


def _normalize(xe_parts, ce_parts):
    return pl.pallas_call(
        _norm_body,
        out_shape=jax.ShapeDtypeStruct((E, D), jnp.float32),
        grid=(E // _BE,),
        in_specs=[
            pl.BlockSpec((NC, _BE, D), lambda i: (0, i, 0)),
            pl.BlockSpec((NC, _BE, D), lambda i: (0, i, 0)),
        ],
        out_specs=pl.BlockSpec((_BE, D), lambda i: (i, 0)),
    )(xe_parts, ce_parts)


def _tail_body(xv_ref, cv_ref, x0_ref, w_ref, ab_ref, out_ref):
    a = ab_ref[0, 0]
    b = ab_ref[0, 1]
    cnt = cv_ref[0, :, 0] + cv_ref[1, :, 0]
    xv = (xv_ref[0] + xv_ref[1]) / jnp.clip(cnt, 1.0)[:, None]
    xi = (1.0 - a) * xv + a * x0_ref[...]
    out_ref[...] = (1.0 - b) * xi + b * jnp.dot(
        xi, w_ref[...].T, preferred_element_type=jnp.float32)


_BN = 1000


def _tail(xv_parts, cv_parts, x0, w, ab):
    return pl.pallas_call(
        _tail_body,
        out_shape=jax.ShapeDtypeStruct((N, D), jnp.float32),
        grid=(N // _BN,),
        in_specs=[
            pl.BlockSpec((NC, _BN, D), lambda i: (0, i, 0)),
            pl.BlockSpec((NC, _BN, D), lambda i: (0, i, 0)),
            pl.BlockSpec((_BN, D), lambda i: (i, 0)),
            pl.BlockSpec((D, D), lambda i: (0, 0)),
            pl.BlockSpec(memory_space=pltpu.SMEM),
        ],
        out_specs=pl.BlockSpec((_BN, D), lambda i: (i, 0)),
    )(xv_parts, cv_parts, x0, w, ab)


def kernel(X, vertex, edges, alpha, beta, X0, W):
    vertex = vertex.astype(jnp.int32)
    edges = edges.astype(jnp.int32)
    z_ed = jnp.zeros((E, D), jnp.float32)
    z_nd = jnp.zeros((N, D), jnp.float32)

    xe_p = _phase1(X, vertex, edges, z_ed)
    ce_p = _ecount(edges, z_ed)
    cv_p = _vcount(vertex, z_nd)
    xe = _normalize(xe_p, ce_p)
    xv_p = _phase2(xe, vertex, edges, z_nd)
    ab = jnp.stack([alpha, beta]).astype(jnp.float32).reshape(1, 2)
    return _tail(xv_p, cv_p, X0, W, ab)
